# baseline (speedup 1.0000x reference)
import functools

import jax
import jax.numpy as jnp
from jax import lax
from jax.experimental import pallas as pl
from jax.experimental.pallas import tpu as pltpu
from jax.experimental.pallas import tpu_sc as plsc

F32 = jnp.float32
I32 = jnp.int32

B, N, T, E = 8, 50000, 12, 800000
HORIZON = 12

NSC = 2
NTILE = 16
NW = NSC * NTILE
ER = E // 128
CHS = 196
NP = 50176
RPS = NP // NTILE
NPR = NP // 16

_mesh = plsc.VectorSubcoreMesh(core_axis_name="c", subcore_axis_name="s")
_sc_params = pltpu.CompilerParams(use_tc_tiling_on_sc=False)


def _worker_span(wid):
    start = wid * ER // NW
    nrows = (wid + 1) * ER // NW - start
    return start, nrows


@functools.partial(
    pl.kernel,
    out_type=(
        jax.ShapeDtypeStruct((NP,), F32),
        jax.ShapeDtypeStruct((NP,), F32),
        jax.ShapeDtypeStruct((NP,), F32),
        jax.ShapeDtypeStruct((NP,), F32),
    ),
    mesh=_mesh,
    scratch_types=[
        pltpu.VMEM((CHS * 128,), I32),
        pltpu.VMEM((CHS * 128,), I32),
        pltpu.VMEM((CHS * 128,), F32),
        pltpu.VMEM((128,), F32),
        pltpu.VMEM((RPS,), F32),
        pltpu.VMEM_SHARED((NP,), F32),
        pltpu.VMEM_SHARED((NP,), F32),
    ],
    compiler_params=_sc_params,
)
def _degrees(row1, col1, w1, ones_h, z1_h, degw0, degw1, cnt0, cnt1,
             rowb, colb, wb, onesv, bounce, degw_s, cnt_s):
    cid = lax.axis_index("c")
    sid = lax.axis_index("s")
    wid = sid * NSC + cid
    off = sid * RPS
    pltpu.sync_copy(z1_h, bounce)
    pltpu.sync_copy(bounce, degw_s.at[pl.ds(off, RPS)])
    pltpu.sync_copy(bounce, cnt_s.at[pl.ds(off, RPS)])
    pltpu.sync_copy(ones_h, onesv)
    start, nrows = _worker_span(wid)
    pltpu.sync_copy(row1.at[pl.ds(start * 128, CHS * 128)], rowb)
    pltpu.sync_copy(col1.at[pl.ds(start * 128, CHS * 128)], colb)
    pltpu.sync_copy(w1.at[pl.ds(start * 128, CHS * 128)], wb)
    plsc.subcore_barrier()

    def body(j, carry):
        pltpu.sync_copy(wb.at[pl.ds(j * 128, 128)],
                        degw_s.at[rowb.at[pl.ds(j * 128, 128)]], add=True)
        pltpu.sync_copy(onesv, cnt_s.at[colb.at[pl.ds(j * 128, 128)]],
                        add=True)
        return carry

    lax.fori_loop(0, nrows, body, 0)
    plsc.subcore_barrier()
    pltpu.sync_copy(degw_s.at[pl.ds(off, RPS)], bounce)

    @pl.when(cid == 0)
    def _():
        pltpu.sync_copy(bounce, degw0.at[pl.ds(off, RPS)])

    @pl.when(cid == 1)
    def _():
        pltpu.sync_copy(bounce, degw1.at[pl.ds(off, RPS)])

    pltpu.sync_copy(cnt_s.at[pl.ds(off, RPS)], bounce)

    @pl.when(cid == 0)
    def _():
        pltpu.sync_copy(bounce, cnt0.at[pl.ds(off, RPS)])

    @pl.when(cid == 1)
    def _():
        pltpu.sync_copy(bounce, cnt1.at[pl.ds(off, RPS)])


NB = 1024
GRID_N = NP // NB


def _xmean_body(x_ref, xm_ref):
    xm_ref[...] = jnp.sum(x_ref[...], axis=2) * (1.0 / T)


_xmean = pl.pallas_call(
    _xmean_body,
    grid=(GRID_N,),
    in_specs=[pl.BlockSpec((B, NB, T), lambda i: (0, i, 0))],
    out_specs=pl.BlockSpec((B, NB), lambda i: (0, i)),
    out_shape=jax.ShapeDtypeStruct((B, NP), F32),
)


def _scale_body(xm_ref, dwa, dwb, ca, cb, xs_ref, dinv_ref):
    iso = 1.0 / (dwa[...] + dwb[...] + 1e-3)
    dinv = lax.rsqrt(ca[...] + cb[...] + 1.0)
    dinv_ref[...] = dinv
    xs_ref[...] = xm_ref[...] * (iso * dinv)[None, :]


_scale = pl.pallas_call(
    _scale_body,
    grid=(GRID_N,),
    in_specs=[pl.BlockSpec((B, NB), lambda i: (0, i))]
    + [pl.BlockSpec((NB,), lambda i: (i,))] * 4,
    out_specs=[
        pl.BlockSpec((B, NB), lambda i: (0, i)),
        pl.BlockSpec((NB,), lambda i: (i,)),
    ],
    out_shape=(
        jax.ShapeDtypeStruct((B, NP), F32),
        jax.ShapeDtypeStruct((NP,), F32),
    ),
)


@functools.partial(
    pl.kernel,
    out_type=(
        jax.ShapeDtypeStruct((NP, B), F32),
        jax.ShapeDtypeStruct((NP, B), F32),
    ),
    mesh=_mesh,
    scratch_types=[
        pltpu.VMEM((CHS * 128,), I32),
        pltpu.VMEM((CHS * 128,), I32),
        pltpu.VMEM((RPS, B), F32),
        pltpu.VMEM((RPS,), F32),
        pltpu.VMEM((128, B), F32),
        pltpu.VMEM_SHARED((NP, B), F32),
        pltpu.VMEM_SHARED((NP, B), F32),
    ],
    compiler_params=_sc_params,
)
def _edgepass(row1, col1, g_h, z8_h, acc0, acc1,
              rowb, colb, gbuf, vb, gath, g_s, acc_s):
    cid = lax.axis_index("c")
    sid = lax.axis_index("s")
    wid = sid * NSC + cid
    off = sid * RPS
    start, nrows = _worker_span(wid)
    pltpu.sync_copy(row1.at[pl.ds(start * 128, CHS * 128)], rowb)
    pltpu.sync_copy(col1.at[pl.ds(start * 128, CHS * 128)], colb)
    pltpu.sync_copy(g_h.at[pl.ds(off, RPS)], gbuf)
    pltpu.sync_copy(gbuf, g_s.at[pl.ds(off, RPS)])

    @pl.when(cid == 0)
    def _():
        pltpu.sync_copy(gbuf, acc_s.at[pl.ds(off, RPS)])

    @pl.when(cid == 1)
    def _():
        pltpu.sync_copy(z8_h, gbuf)
        pltpu.sync_copy(gbuf, acc_s.at[pl.ds(off, RPS)])

    plsc.subcore_barrier()

    def ebody(j, carry):
        pltpu.sync_copy(g_s.at[rowb.at[pl.ds(j * 128, 128)]], gath)
        pltpu.sync_copy(gath, acc_s.at[colb.at[pl.ds(j * 128, 128)]],
                        add=True)
        return carry

    lax.fori_loop(0, nrows, ebody, 0)
    plsc.subcore_barrier()
    pltpu.sync_copy(acc_s.at[pl.ds(off, RPS)], gbuf)

    @pl.when(cid == 0)
    def _():
        pltpu.sync_copy(gbuf, acc0.at[pl.ds(off, RPS)])

    @pl.when(cid == 1)
    def _():
        pltpu.sync_copy(gbuf, acc1.at[pl.ds(off, RPS)])


def _fin_body(ta, dinv_ref, w_ref, b_ref, pw_ref, pb_ref, out_ref):
    tt = ta[...] * dinv_ref[...][None, :]
    v = jnp.dot(w_ref[0, :], pw_ref[...], preferred_element_type=F32)
    c = jnp.dot(b_ref[...], pw_ref[...], preferred_element_type=F32) + pb_ref[...]
    out_ref[...] = tt[:, :, None] * v[None, None, :] + c[None, None, :]


_fin = pl.pallas_call(
    _fin_body,
    grid=(GRID_N,),
    in_specs=[
        pl.BlockSpec((B, NB), lambda i: (0, i)),
        pl.BlockSpec((NB,), lambda i: (i,)),
        pl.BlockSpec((1, 64), lambda i: (0, 0)),
        pl.BlockSpec((64,), lambda i: (0,)),
        pl.BlockSpec((64, HORIZON), lambda i: (0, 0)),
        pl.BlockSpec((HORIZON,), lambda i: (0,)),
    ],
    out_specs=pl.BlockSpec((B, NB, HORIZON), lambda i: (0, i, 0)),
    out_shape=jax.ShapeDtypeStruct((B, N, HORIZON), F32),
)


def kernel(x, edge_index, edge_weight, W, b, proj_W, proj_b):
    x3 = x[..., 0]
    row1 = edge_index[0]
    col1 = edge_index[1]
    ones_h = jnp.ones((128,), F32)
    z1 = jnp.zeros((RPS,), F32)
    z8 = jnp.zeros((RPS, B), F32)
    degw0, degw1, cnt0, cnt1 = _degrees(row1, col1, edge_weight, ones_h, z1)
    xm = _xmean(x3)
    xs, dinv = _scale(xm, degw0, degw1, cnt0, cnt1)
    g = jnp.transpose(xs)
    acc0, acc1 = _edgepass(row1, col1, g, z8)
    asum = jnp.transpose(acc0 + acc1)
    return _fin(asum, dinv, W, b, proj_W, proj_b)


# speedup vs baseline: 1.6640x; 1.6640x over previous
import functools

import jax
import jax.numpy as jnp
from jax import lax
from jax.experimental import pallas as pl
from jax.experimental.pallas import tpu as pltpu
from jax.experimental.pallas import tpu_sc as plsc

F32 = jnp.float32
I32 = jnp.int32

B, N, T, E = 8, 50000, 12, 800000
HORIZON = 12

NSC = 2
NTILE = 16
NW = NSC * NTILE
ER = E // 128
CHS = 196
NP = 50176
RPS = NP // NTILE
NPR = NP // 16

_mesh = plsc.VectorSubcoreMesh(core_axis_name="c", subcore_axis_name="s")
_sc_params = pltpu.CompilerParams(use_tc_tiling_on_sc=False)


def _worker_span(wid):
    start = wid * ER // NW
    nrows = (wid + 1) * ER // NW - start
    return start, nrows


EB = 131072
EGRID = -(-E // EB)


def _split_body(idx_ref, row_ref, col_ref):
    row_ref[...] = idx_ref[0]
    col_ref[...] = idx_ref[1]


_split = pl.pallas_call(
    _split_body,
    grid=(EGRID,),
    in_specs=[pl.BlockSpec((2, EB), lambda i: (0, i))],
    out_specs=[
        pl.BlockSpec((EB,), lambda i: (i,)),
        pl.BlockSpec((EB,), lambda i: (i,)),
    ],
    out_shape=(
        jax.ShapeDtypeStruct((E,), I32),
        jax.ShapeDtypeStruct((E,), I32),
    ),
)

@functools.partial(
    pl.kernel,
    out_type=(
        jax.ShapeDtypeStruct((NP,), F32),
        jax.ShapeDtypeStruct((NP,), F32),
        jax.ShapeDtypeStruct((NP,), F32),
        jax.ShapeDtypeStruct((NP,), F32),
    ),
    mesh=_mesh,
    scratch_types=[
        pltpu.VMEM((CHS * 128,), I32),
        pltpu.VMEM((CHS * 128,), I32),
        pltpu.VMEM((CHS * 128,), F32),
        pltpu.VMEM((128,), F32),
        pltpu.VMEM((RPS,), F32),
        pltpu.VMEM_SHARED((NP,), F32),
        pltpu.VMEM_SHARED((NP,), F32),
    ],
    compiler_params=_sc_params,
)
def _degrees(row1, col1, w1, ones_h, z1_h, degw0, degw1, cnt0, cnt1,
             rowb, colb, wb, onesv, bounce, degw_s, cnt_s):
    cid = lax.axis_index("c")
    sid = lax.axis_index("s")
    wid = sid * NSC + cid
    off = sid * RPS
    pltpu.sync_copy(z1_h, bounce)
    pltpu.sync_copy(bounce, degw_s.at[pl.ds(off, RPS)])
    pltpu.sync_copy(bounce, cnt_s.at[pl.ds(off, RPS)])
    pltpu.sync_copy(ones_h, onesv)
    start, nrows = _worker_span(wid)
    pltpu.sync_copy(row1.at[pl.ds(start * 128, CHS * 128)], rowb)
    pltpu.sync_copy(col1.at[pl.ds(start * 128, CHS * 128)], colb)
    pltpu.sync_copy(w1.at[pl.ds(start * 128, CHS * 128)], wb)
    plsc.subcore_barrier()

    def body(j, carry):
        pltpu.sync_copy(wb.at[pl.ds(j * 128, 128)],
                        degw_s.at[rowb.at[pl.ds(j * 128, 128)]], add=True)
        pltpu.sync_copy(onesv, cnt_s.at[colb.at[pl.ds(j * 128, 128)]],
                        add=True)
        return carry

    lax.fori_loop(0, nrows, body, 0)
    plsc.subcore_barrier()
    pltpu.sync_copy(degw_s.at[pl.ds(off, RPS)], bounce)

    @pl.when(cid == 0)
    def _():
        pltpu.sync_copy(bounce, degw0.at[pl.ds(off, RPS)])

    @pl.when(cid == 1)
    def _():
        pltpu.sync_copy(bounce, degw1.at[pl.ds(off, RPS)])

    pltpu.sync_copy(cnt_s.at[pl.ds(off, RPS)], bounce)

    @pl.when(cid == 0)
    def _():
        pltpu.sync_copy(bounce, cnt0.at[pl.ds(off, RPS)])

    @pl.when(cid == 1)
    def _():
        pltpu.sync_copy(bounce, cnt1.at[pl.ds(off, RPS)])


NB = 1024
GRID_N = NP // NB


def _xmean_body(x_ref, xm_ref):
    xm_ref[...] = jnp.sum(x_ref[...], axis=0) * (1.0 / T)


_xmean = pl.pallas_call(
    _xmean_body,
    grid=(GRID_N,),
    in_specs=[pl.BlockSpec((T, B, NB), lambda i: (0, 0, i))],
    out_specs=pl.BlockSpec((B, NB), lambda i: (0, i)),
    out_shape=jax.ShapeDtypeStruct((B, NP), F32),
)


def _scale_body(xm_ref, dwa, dwb, ca, cb, xs_ref, dinv_ref):
    iso = 1.0 / (dwa[...] + dwb[...] + 1e-3)
    dinv = lax.rsqrt(ca[...] + cb[...] + 1.0)
    dinv_ref[...] = dinv
    xs_ref[...] = xm_ref[...] * (iso * dinv)[None, :]


_scale = pl.pallas_call(
    _scale_body,
    grid=(GRID_N,),
    in_specs=[pl.BlockSpec((B, NB), lambda i: (0, i))]
    + [pl.BlockSpec((NB,), lambda i: (i,))] * 4,
    out_specs=[
        pl.BlockSpec((B, NB), lambda i: (0, i)),
        pl.BlockSpec((NB,), lambda i: (i,)),
    ],
    out_shape=(
        jax.ShapeDtypeStruct((B, NP), F32),
        jax.ShapeDtypeStruct((NP,), F32),
    ),
)


@functools.partial(
    pl.kernel,
    out_type=(
        jax.ShapeDtypeStruct((NP, B), F32),
        jax.ShapeDtypeStruct((NP, B), F32),
    ),
    mesh=_mesh,
    scratch_types=[
        pltpu.VMEM((CHS * 128,), I32),
        pltpu.VMEM((CHS * 128,), I32),
        pltpu.VMEM((RPS, B), F32),
        pltpu.VMEM((RPS,), F32),
        pltpu.VMEM((128, B), F32),
        pltpu.VMEM_SHARED((NP, B), F32),
        pltpu.VMEM_SHARED((NP, B), F32),
    ],
    compiler_params=_sc_params,
)
def _edgepass(row1, col1, g_h, z8_h, acc0, acc1,
              rowb, colb, gbuf, vb, gath, g_s, acc_s):
    cid = lax.axis_index("c")
    sid = lax.axis_index("s")
    wid = sid * NSC + cid
    off = sid * RPS
    start, nrows = _worker_span(wid)
    pltpu.sync_copy(row1.at[pl.ds(start * 128, CHS * 128)], rowb)
    pltpu.sync_copy(col1.at[pl.ds(start * 128, CHS * 128)], colb)
    pltpu.sync_copy(g_h.at[pl.ds(off, RPS)], gbuf)
    pltpu.sync_copy(gbuf, g_s.at[pl.ds(off, RPS)])

    @pl.when(cid == 0)
    def _():
        pltpu.sync_copy(gbuf, acc_s.at[pl.ds(off, RPS)])

    @pl.when(cid == 1)
    def _():
        pltpu.sync_copy(z8_h, gbuf)
        pltpu.sync_copy(gbuf, acc_s.at[pl.ds(off, RPS)])

    plsc.subcore_barrier()

    def ebody(j, carry):
        pltpu.sync_copy(g_s.at[rowb.at[pl.ds(j * 128, 128)]], gath)
        pltpu.sync_copy(gath, acc_s.at[colb.at[pl.ds(j * 128, 128)]],
                        add=True)
        return carry

    lax.fori_loop(0, nrows, ebody, 0)
    plsc.subcore_barrier()
    pltpu.sync_copy(acc_s.at[pl.ds(off, RPS)], gbuf)

    @pl.when(cid == 0)
    def _():
        pltpu.sync_copy(gbuf, acc0.at[pl.ds(off, RPS)])

    @pl.when(cid == 1)
    def _():
        pltpu.sync_copy(gbuf, acc1.at[pl.ds(off, RPS)])


def _fin_body(ta, dinv_ref, w_ref, b_ref, pw_ref, pb_ref, *out_refs):
    tt = ta[...] * dinv_ref[...][None, :]
    v = jnp.dot(w_ref[0, :], pw_ref[...], preferred_element_type=F32)
    c = jnp.dot(b_ref[...], pw_ref[...], preferred_element_type=F32) + pb_ref[...]
    for h in range(HORIZON):
        out_refs[h][...] = tt * v[h] + c[h]


_fin = pl.pallas_call(
    _fin_body,
    grid=(GRID_N,),
    in_specs=[
        pl.BlockSpec((B, NB), lambda i: (0, i)),
        pl.BlockSpec((NB,), lambda i: (i,)),
        pl.BlockSpec((1, 64), lambda i: (0, 0)),
        pl.BlockSpec((64,), lambda i: (0,)),
        pl.BlockSpec((64, HORIZON), lambda i: (0, 0)),
        pl.BlockSpec((HORIZON,), lambda i: (0,)),
    ],
    out_specs=[pl.BlockSpec((B, NB), lambda i: (0, i))] * HORIZON,
    out_shape=tuple(jax.ShapeDtypeStruct((B, NP), F32)
                    for _ in range(HORIZON)),
)


def kernel(x, edge_index, edge_weight, W, b, proj_W, proj_b):
    xt = jnp.transpose(x.reshape(B, N, T), (2, 0, 1))
    row1, col1 = _split(edge_index)
    ones_h = jnp.ones((128,), F32)
    z1 = jnp.zeros((RPS,), F32)
    z8 = jnp.zeros((RPS, B), F32)
    degw0, degw1, cnt0, cnt1 = _degrees(row1, col1, edge_weight, ones_h, z1)
    xm = _xmean(xt)
    xs, dinv = _scale(xm, degw0, degw1, cnt0, cnt1)
    g = jnp.transpose(xs)
    acc0, acc1 = _edgepass(row1, col1, g, z8)
    asum = jnp.transpose(acc0 + acc1)
    planes = _fin(asum, dinv, W, b, proj_W, proj_b)
    return jnp.stack([p[:, :N] for p in planes], axis=-1)


# speedup vs baseline: 2.1401x; 1.2861x over previous
import functools

import jax
import jax.numpy as jnp
from jax import lax
from jax.experimental import pallas as pl
from jax.experimental.pallas import tpu as pltpu
from jax.experimental.pallas import tpu_sc as plsc

F32 = jnp.float32
I32 = jnp.int32

B, N, T, E = 8, 50000, 12, 800000
HORIZON = 12

NSC = 2
NTILE = 16
NW = NSC * NTILE
ER = E // 128
CHS = 196
NP = 50176
RPS = NP // NTILE
NPR = NP // 16

_mesh = plsc.VectorSubcoreMesh(core_axis_name="c", subcore_axis_name="s")
_sc_params = pltpu.CompilerParams(use_tc_tiling_on_sc=False)


def _worker_span(wid):
    start = wid * ER // NW
    nrows = (wid + 1) * ER // NW - start
    return start, nrows


EB = 131072
EGRID = -(-E // EB)


def _split_body(idx_ref, row_ref, col_ref):
    row_ref[...] = idx_ref[0]
    col_ref[...] = idx_ref[1]


_split = pl.pallas_call(
    _split_body,
    grid=(EGRID,),
    in_specs=[pl.BlockSpec((2, EB), lambda i: (0, i))],
    out_specs=[
        pl.BlockSpec((EB,), lambda i: (i,)),
        pl.BlockSpec((EB,), lambda i: (i,)),
    ],
    out_shape=(
        jax.ShapeDtypeStruct((E,), I32),
        jax.ShapeDtypeStruct((E,), I32),
    ),
)

@functools.partial(
    pl.kernel,
    out_type=(
        jax.ShapeDtypeStruct((NP,), F32),
        jax.ShapeDtypeStruct((NP,), F32),
        jax.ShapeDtypeStruct((NP,), F32),
        jax.ShapeDtypeStruct((NP,), F32),
    ),
    mesh=_mesh,
    scratch_types=[
        pltpu.VMEM((CHS * 128,), I32),
        pltpu.VMEM((CHS * 128,), I32),
        pltpu.VMEM((CHS * 128,), F32),
        pltpu.VMEM((128,), F32),
        pltpu.VMEM((RPS,), F32),
        pltpu.VMEM_SHARED((NP,), F32),
        pltpu.VMEM_SHARED((NP,), F32),
        pltpu.SemaphoreType.DMA,
        pltpu.SemaphoreType.DMA,
    ],
    compiler_params=_sc_params,
)
def _degrees(row1, col1, w1, ones_h, z1_h, degw0, degw1, cnt0, cnt1,
             rowb, colb, wb, onesv, bounce, degw_s, cnt_s, sma, smb):
    cid = lax.axis_index("c")
    sid = lax.axis_index("s")
    wid = sid * NSC + cid
    off = sid * RPS
    pltpu.sync_copy(z1_h, bounce)
    pltpu.sync_copy(bounce, degw_s.at[pl.ds(off, RPS)])
    pltpu.sync_copy(bounce, cnt_s.at[pl.ds(off, RPS)])
    pltpu.sync_copy(ones_h, onesv)
    start, nrows = _worker_span(wid)
    pltpu.sync_copy(row1.at[pl.ds(start * 128, CHS * 128)], rowb)
    pltpu.sync_copy(col1.at[pl.ds(start * 128, CHS * 128)], colb)
    pltpu.sync_copy(w1.at[pl.ds(start * 128, CHS * 128)], wb)
    plsc.subcore_barrier()

    def body(j, carry):
        d1 = pltpu.async_copy(wb.at[pl.ds(j * 128, 128)],
                              degw_s.at[rowb.at[pl.ds(j * 128, 128)]], sma,
                              add=True)
        d2 = pltpu.async_copy(onesv, cnt_s.at[colb.at[pl.ds(j * 128, 128)]],
                              smb, add=True)
        d1.wait()
        d2.wait()
        return carry

    lax.fori_loop(0, nrows, body, 0)
    plsc.subcore_barrier()
    pltpu.sync_copy(degw_s.at[pl.ds(off, RPS)], bounce)

    @pl.when(cid == 0)
    def _():
        pltpu.sync_copy(bounce, degw0.at[pl.ds(off, RPS)])

    @pl.when(cid == 1)
    def _():
        pltpu.sync_copy(bounce, degw1.at[pl.ds(off, RPS)])

    pltpu.sync_copy(cnt_s.at[pl.ds(off, RPS)], bounce)

    @pl.when(cid == 0)
    def _():
        pltpu.sync_copy(bounce, cnt0.at[pl.ds(off, RPS)])

    @pl.when(cid == 1)
    def _():
        pltpu.sync_copy(bounce, cnt1.at[pl.ds(off, RPS)])


NB = 7168
GRID_N = NP // NB


def _xmean_body(x_ref, xm_ref):
    xm_ref[...] = jnp.sum(x_ref[...], axis=0) * (1.0 / T)


_xmean = pl.pallas_call(
    _xmean_body,
    grid=(GRID_N,),
    in_specs=[pl.BlockSpec((T, B, NB), lambda i: (0, 0, i))],
    out_specs=pl.BlockSpec((B, NB), lambda i: (0, i)),
    out_shape=jax.ShapeDtypeStruct((B, NP), F32),
)


def _scale_body(xm_ref, dwa, dwb, ca, cb, xs_ref, dinv_ref):
    iso = 1.0 / (dwa[...] + dwb[...] + 1e-3)
    dinv = lax.rsqrt(ca[...] + cb[...] + 1.0)
    dinv_ref[...] = dinv
    xs_ref[...] = xm_ref[...] * (iso * dinv)[None, :]


_scale = pl.pallas_call(
    _scale_body,
    grid=(GRID_N,),
    in_specs=[pl.BlockSpec((B, NB), lambda i: (0, i))]
    + [pl.BlockSpec((NB,), lambda i: (i,))] * 4,
    out_specs=[
        pl.BlockSpec((B, NB), lambda i: (0, i)),
        pl.BlockSpec((NB,), lambda i: (i,)),
    ],
    out_shape=(
        jax.ShapeDtypeStruct((B, NP), F32),
        jax.ShapeDtypeStruct((NP,), F32),
    ),
)


@functools.partial(
    pl.kernel,
    out_type=(
        jax.ShapeDtypeStruct((NP, B), F32),
        jax.ShapeDtypeStruct((NP, B), F32),
    ),
    mesh=_mesh,
    scratch_types=[
        pltpu.VMEM((CHS * 128,), I32),
        pltpu.VMEM((CHS * 128,), I32),
        pltpu.VMEM((RPS, B), F32),
        pltpu.VMEM((128, B), F32),
        pltpu.VMEM((128, B), F32),
        pltpu.VMEM_SHARED((NP, B), F32),
        pltpu.VMEM_SHARED((NP, B), F32),
        pltpu.SemaphoreType.DMA,
        pltpu.SemaphoreType.DMA,
    ],
    compiler_params=_sc_params,
)
def _edgepass(row1, col1, g_h, z8_h, acc0, acc1,
              rowb, colb, gbuf, gath0, gath1, g_s, acc_s, sg0, sg1):
    cid = lax.axis_index("c")
    sid = lax.axis_index("s")
    wid = sid * NSC + cid
    off = sid * RPS
    start, nrows = _worker_span(wid)
    pltpu.sync_copy(row1.at[pl.ds(start * 128, CHS * 128)], rowb)
    pltpu.sync_copy(col1.at[pl.ds(start * 128, CHS * 128)], colb)
    pltpu.sync_copy(g_h.at[pl.ds(off, RPS)], gbuf)
    pltpu.sync_copy(gbuf, g_s.at[pl.ds(off, RPS)])

    @pl.when(cid == 0)
    def _():
        pltpu.sync_copy(gbuf, acc_s.at[pl.ds(off, RPS)])

    @pl.when(cid == 1)
    def _():
        pltpu.sync_copy(z8_h, gbuf)
        pltpu.sync_copy(gbuf, acc_s.at[pl.ds(off, RPS)])

    plsc.subcore_barrier()

    pltpu.async_copy(g_s.at[rowb.at[pl.ds(0, 128)]], gath0, sg0)

    def pbody(k, carry):
        j0 = 2 * k
        j1 = j0 + 1
        pltpu.async_copy(g_s.at[rowb.at[pl.ds(j1 * 128, 128)]], gath1, sg1)
        pltpu.make_async_copy(
            g_s.at[rowb.at[pl.ds(j0 * 128, 128)]], gath0, sg0).wait()
        pltpu.sync_copy(gath0, acc_s.at[colb.at[pl.ds(j0 * 128, 128)]],
                        add=True)

        @pl.when(j0 + 2 < nrows)
        def _():
            pltpu.async_copy(
                g_s.at[rowb.at[pl.ds((j0 + 2) * 128, 128)]], gath0, sg0)

        pltpu.make_async_copy(
            g_s.at[rowb.at[pl.ds(j1 * 128, 128)]], gath1, sg1).wait()
        pltpu.sync_copy(gath1, acc_s.at[colb.at[pl.ds(j1 * 128, 128)]],
                        add=True)
        return carry

    lax.fori_loop(0, nrows // 2, pbody, 0)

    @pl.when(nrows % 2 == 1)
    def _():
        j = nrows - 1
        pltpu.make_async_copy(
            g_s.at[rowb.at[pl.ds(j * 128, 128)]], gath0, sg0).wait()
        pltpu.sync_copy(gath0, acc_s.at[colb.at[pl.ds(j * 128, 128)]],
                        add=True)
    plsc.subcore_barrier()
    pltpu.sync_copy(acc_s.at[pl.ds(off, RPS)], gbuf)

    @pl.when(cid == 0)
    def _():
        pltpu.sync_copy(gbuf, acc0.at[pl.ds(off, RPS)])

    @pl.when(cid == 1)
    def _():
        pltpu.sync_copy(gbuf, acc1.at[pl.ds(off, RPS)])


def _fin_body(ta, dinv_ref, w_ref, b_ref, pw_ref, pb_ref, *out_refs):
    tt = ta[...] * dinv_ref[...][None, :]
    v = jnp.dot(w_ref[0, :], pw_ref[...], preferred_element_type=F32)
    c = jnp.dot(b_ref[...], pw_ref[...], preferred_element_type=F32) + pb_ref[...]
    for h in range(HORIZON):
        out_refs[h][...] = tt * v[h] + c[h]


_fin = pl.pallas_call(
    _fin_body,
    grid=(GRID_N,),
    in_specs=[
        pl.BlockSpec((B, NB), lambda i: (0, i)),
        pl.BlockSpec((NB,), lambda i: (i,)),
        pl.BlockSpec((1, 64), lambda i: (0, 0)),
        pl.BlockSpec((64,), lambda i: (0,)),
        pl.BlockSpec((64, HORIZON), lambda i: (0, 0)),
        pl.BlockSpec((HORIZON,), lambda i: (0,)),
    ],
    out_specs=[pl.BlockSpec((B, NB), lambda i: (0, i))] * HORIZON,
    out_shape=tuple(jax.ShapeDtypeStruct((B, NP), F32)
                    for _ in range(HORIZON)),
)


def kernel(x, edge_index, edge_weight, W, b, proj_W, proj_b):
    xt = jnp.transpose(x.reshape(B, N, T), (2, 0, 1))
    row1, col1 = _split(edge_index)
    ones_h = jnp.ones((128,), F32)
    z1 = jnp.zeros((RPS,), F32)
    z8 = jnp.zeros((RPS, B), F32)
    degw0, degw1, cnt0, cnt1 = _degrees(row1, col1, edge_weight, ones_h, z1)
    xm = _xmean(xt)
    xs, dinv = _scale(xm, degw0, degw1, cnt0, cnt1)
    g = jnp.transpose(xs)
    acc0, acc1 = _edgepass(row1, col1, g, z8)
    asum = jnp.transpose(acc0 + acc1)
    planes = _fin(asum, dinv, W, b, proj_W, proj_b)
    return jnp.stack([p[:, :N] for p in planes], axis=-1)


# speedup vs baseline: 2.2593x; 1.0557x over previous
import functools

import jax
import jax.numpy as jnp
from jax import lax
from jax.experimental import pallas as pl
from jax.experimental.pallas import tpu as pltpu
from jax.experimental.pallas import tpu_sc as plsc

F32 = jnp.float32
I32 = jnp.int32

B, N, T, E = 8, 50000, 12, 800000
HORIZON = 12

NSC = 2
NTILE = 16
NW = NSC * NTILE
ER = E // 128
CHS = 196
NP = 50176
RPS = NP // NTILE
NPR = NP // 16

_mesh = plsc.VectorSubcoreMesh(core_axis_name="c", subcore_axis_name="s")
_sc_params = pltpu.CompilerParams(use_tc_tiling_on_sc=False)


def _worker_span(wid):
    start = wid * ER // NW
    nrows = (wid + 1) * ER // NW - start
    return start, nrows


EB = 131072
EGRID = -(-E // EB)


def _split_body(idx_ref, row_ref, col_ref):
    row_ref[...] = idx_ref[0]
    col_ref[...] = idx_ref[1]


_split = pl.pallas_call(
    _split_body,
    grid=(EGRID,),
    in_specs=[pl.BlockSpec((2, EB), lambda i: (0, i))],
    out_specs=[
        pl.BlockSpec((EB,), lambda i: (i,)),
        pl.BlockSpec((EB,), lambda i: (i,)),
    ],
    out_shape=(
        jax.ShapeDtypeStruct((E,), I32),
        jax.ShapeDtypeStruct((E,), I32),
    ),
)

@functools.partial(
    pl.kernel,
    out_type=(
        jax.ShapeDtypeStruct((NP,), F32),
        jax.ShapeDtypeStruct((NP,), F32),
        jax.ShapeDtypeStruct((NP,), F32),
        jax.ShapeDtypeStruct((NP,), F32),
    ),
    mesh=_mesh,
    scratch_types=[
        pltpu.VMEM((CHS * 128,), I32),
        pltpu.VMEM((CHS * 128,), I32),
        pltpu.VMEM((CHS * 128,), F32),
        pltpu.VMEM((128,), F32),
        pltpu.VMEM((RPS,), F32),
        pltpu.VMEM_SHARED((NP,), F32),
        pltpu.VMEM_SHARED((NP,), F32),
        pltpu.SemaphoreType.DMA,
        pltpu.SemaphoreType.DMA,
    ],
    compiler_params=_sc_params,
)
def _degrees(row1, col1, w1, ones_h, z1_h, degw0, degw1, cnt0, cnt1,
             rowb, colb, wb, onesv, bounce, degw_s, cnt_s, sma, smb):
    cid = lax.axis_index("c")
    sid = lax.axis_index("s")
    wid = sid * NSC + cid
    off = sid * RPS
    pltpu.sync_copy(z1_h, bounce)
    pltpu.sync_copy(bounce, degw_s.at[pl.ds(off, RPS)])
    pltpu.sync_copy(bounce, cnt_s.at[pl.ds(off, RPS)])
    pltpu.sync_copy(ones_h, onesv)
    start, nrows = _worker_span(wid)
    pltpu.sync_copy(row1.at[pl.ds(start * 128, CHS * 128)], rowb)
    pltpu.sync_copy(col1.at[pl.ds(start * 128, CHS * 128)], colb)
    pltpu.sync_copy(w1.at[pl.ds(start * 128, CHS * 128)], wb)
    plsc.subcore_barrier()

    pltpu.async_copy(wb.at[pl.ds(0, 128)],
                     degw_s.at[rowb.at[pl.ds(0, 128)]], sma, add=True)
    pltpu.async_copy(onesv, cnt_s.at[colb.at[pl.ds(0, 128)]], smb, add=True)

    def body(j, carry):
        @pl.when(j + 1 < nrows)
        def _():
            pltpu.async_copy(wb.at[pl.ds((j + 1) * 128, 128)],
                             degw_s.at[rowb.at[pl.ds((j + 1) * 128, 128)]],
                             sma, add=True)
            pltpu.async_copy(onesv,
                             cnt_s.at[colb.at[pl.ds((j + 1) * 128, 128)]],
                             smb, add=True)

        pltpu.make_async_copy(wb.at[pl.ds(j * 128, 128)],
                              degw_s.at[rowb.at[pl.ds(j * 128, 128)]],
                              sma).wait()
        pltpu.make_async_copy(onesv,
                              cnt_s.at[colb.at[pl.ds(j * 128, 128)]],
                              smb).wait()
        return carry

    lax.fori_loop(0, nrows, body, 0)
    plsc.subcore_barrier()
    pltpu.sync_copy(degw_s.at[pl.ds(off, RPS)], bounce)

    @pl.when(cid == 0)
    def _():
        pltpu.sync_copy(bounce, degw0.at[pl.ds(off, RPS)])

    @pl.when(cid == 1)
    def _():
        pltpu.sync_copy(bounce, degw1.at[pl.ds(off, RPS)])

    pltpu.sync_copy(cnt_s.at[pl.ds(off, RPS)], bounce)

    @pl.when(cid == 0)
    def _():
        pltpu.sync_copy(bounce, cnt0.at[pl.ds(off, RPS)])

    @pl.when(cid == 1)
    def _():
        pltpu.sync_copy(bounce, cnt1.at[pl.ds(off, RPS)])


NB = 7168
GRID_N = NP // NB


def _xmean_body(x_ref, xm_ref):
    xm_ref[...] = jnp.sum(x_ref[...], axis=0) * (1.0 / T)


_xmean = pl.pallas_call(
    _xmean_body,
    grid=(GRID_N,),
    in_specs=[pl.BlockSpec((T, B, NB), lambda i: (0, 0, i))],
    out_specs=pl.BlockSpec((B, NB), lambda i: (0, i)),
    out_shape=jax.ShapeDtypeStruct((B, NP), F32),
)


def _scale_body(xm_ref, dwa, dwb, ca, cb, xs_ref, dinv_ref):
    iso = 1.0 / (dwa[...] + dwb[...] + 1e-3)
    dinv = lax.rsqrt(ca[...] + cb[...] + 1.0)
    dinv_ref[...] = dinv
    xs_ref[...] = xm_ref[...] * (iso * dinv)[None, :]


_scale = pl.pallas_call(
    _scale_body,
    grid=(GRID_N,),
    in_specs=[pl.BlockSpec((B, NB), lambda i: (0, i))]
    + [pl.BlockSpec((NB,), lambda i: (i,))] * 4,
    out_specs=[
        pl.BlockSpec((B, NB), lambda i: (0, i)),
        pl.BlockSpec((NB,), lambda i: (i,)),
    ],
    out_shape=(
        jax.ShapeDtypeStruct((B, NP), F32),
        jax.ShapeDtypeStruct((NP,), F32),
    ),
)


@functools.partial(
    pl.kernel,
    out_type=(
        jax.ShapeDtypeStruct((NP, B), F32),
        jax.ShapeDtypeStruct((NP, B), F32),
    ),
    mesh=_mesh,
    scratch_types=[
        pltpu.VMEM((CHS * 128,), I32),
        pltpu.VMEM((CHS * 128,), I32),
        pltpu.VMEM((RPS, B), F32),
    ] + [pltpu.VMEM((128, B), F32)] * 4
    + [pltpu.VMEM_SHARED((NP, B), F32),
        pltpu.VMEM_SHARED((NP, B), F32),
    ] + [pltpu.SemaphoreType.DMA] * 8,
    compiler_params=_sc_params,
)
def _edgepass(row1, col1, g_h, z8_h, acc0, acc1,
              rowb, colb, gbuf, g0, g1, g2, g3,
              g_s, acc_s, *sems):
    cid = lax.axis_index("c")
    sid = lax.axis_index("s")
    wid = sid * NSC + cid
    off = sid * RPS
    start, nrows = _worker_span(wid)
    pltpu.sync_copy(row1.at[pl.ds(start * 128, CHS * 128)], rowb)
    pltpu.sync_copy(col1.at[pl.ds(start * 128, CHS * 128)], colb)
    pltpu.sync_copy(g_h.at[pl.ds(off, RPS)], gbuf)
    pltpu.sync_copy(gbuf, g_s.at[pl.ds(off, RPS)])

    @pl.when(cid == 0)
    def _():
        pltpu.sync_copy(gbuf, acc_s.at[pl.ds(off, RPS)])

    @pl.when(cid == 1)
    def _():
        pltpu.sync_copy(z8_h, gbuf)
        pltpu.sync_copy(gbuf, acc_s.at[pl.ds(off, RPS)])

    plsc.subcore_barrier()

    gbufs = (g0, g1, g2, g3)
    sg = sems[:4]
    ss = sems[4:]
    NBUF = 4
    for i in range(NBUF):
        pltpu.async_copy(g_s.at[rowb.at[pl.ds(i * 128, 128)]], gbufs[i],
                         sg[i])

    def obody(k, carry):
        j = k * NBUF
        for i in range(NBUF):
            pltpu.make_async_copy(
                g_s.at[rowb.at[pl.ds((j + i) * 128, 128)]], gbufs[i],
                sg[i]).wait()
            pltpu.async_copy(
                gbufs[i], acc_s.at[colb.at[pl.ds((j + i) * 128, 128)]],
                ss[i], add=True)
        for i in range(NBUF):
            pltpu.make_async_copy(
                gbufs[i], acc_s.at[colb.at[pl.ds((j + i) * 128, 128)]],
                ss[i]).wait()
            jn = j + NBUF + i

            @pl.when(jn < nrows)
            def _(jn=jn, i=i):
                pltpu.async_copy(
                    g_s.at[rowb.at[pl.ds(jn * 128, 128)]], gbufs[i], sg[i])
        return carry

    lax.fori_loop(0, nrows // NBUF, obody, 0)

    tbase = nrows // NBUF * NBUF
    for i in range(NBUF - 1):
        jt = tbase + i

        @pl.when(jt < nrows)
        def _(jt=jt, i=i):
            pltpu.make_async_copy(
                g_s.at[rowb.at[pl.ds(jt * 128, 128)]], gbufs[i],
                sg[i]).wait()
            pltpu.sync_copy(
                gbufs[i], acc_s.at[colb.at[pl.ds(jt * 128, 128)]], add=True)
    plsc.subcore_barrier()
    pltpu.sync_copy(acc_s.at[pl.ds(off, RPS)], gbuf)

    @pl.when(cid == 0)
    def _():
        pltpu.sync_copy(gbuf, acc0.at[pl.ds(off, RPS)])

    @pl.when(cid == 1)
    def _():
        pltpu.sync_copy(gbuf, acc1.at[pl.ds(off, RPS)])


def _fin_body(ta, dinv_ref, w_ref, b_ref, pw_ref, pb_ref, *out_refs):
    tt = ta[...] * dinv_ref[...][None, :]
    v = jnp.dot(w_ref[0, :], pw_ref[...], preferred_element_type=F32)
    c = jnp.dot(b_ref[...], pw_ref[...], preferred_element_type=F32) + pb_ref[...]
    for h in range(HORIZON):
        out_refs[h][...] = tt * v[h] + c[h]


_fin = pl.pallas_call(
    _fin_body,
    grid=(GRID_N,),
    in_specs=[
        pl.BlockSpec((B, NB), lambda i: (0, i)),
        pl.BlockSpec((NB,), lambda i: (i,)),
        pl.BlockSpec((1, 64), lambda i: (0, 0)),
        pl.BlockSpec((64,), lambda i: (0,)),
        pl.BlockSpec((64, HORIZON), lambda i: (0, 0)),
        pl.BlockSpec((HORIZON,), lambda i: (0,)),
    ],
    out_specs=[pl.BlockSpec((B, NB), lambda i: (0, i))] * HORIZON,
    out_shape=tuple(jax.ShapeDtypeStruct((B, NP), F32)
                    for _ in range(HORIZON)),
)


def kernel(x, edge_index, edge_weight, W, b, proj_W, proj_b):
    xt = jnp.transpose(x.reshape(B, N, T), (2, 0, 1))
    row1, col1 = _split(edge_index)
    ones_h = jnp.ones((128,), F32)
    z1 = jnp.zeros((RPS,), F32)
    z8 = jnp.zeros((RPS, B), F32)
    degw0, degw1, cnt0, cnt1 = _degrees(row1, col1, edge_weight, ones_h, z1)
    xm = _xmean(xt)
    xs, dinv = _scale(xm, degw0, degw1, cnt0, cnt1)
    g = jnp.transpose(xs)
    acc0, acc1 = _edgepass(row1, col1, g, z8)
    asum = jnp.transpose(acc0 + acc1)
    planes = _fin(asum, dinv, W, b, proj_W, proj_b)
    return jnp.stack([p[:, :N] for p in planes], axis=-1)


# speedup vs baseline: 2.4216x; 1.0718x over previous
import functools

import jax
import jax.numpy as jnp
from jax import lax
from jax.experimental import pallas as pl
from jax.experimental.pallas import tpu as pltpu
from jax.experimental.pallas import tpu_sc as plsc

F32 = jnp.float32
I32 = jnp.int32

B, N, T, E = 8, 50000, 12, 800000
HORIZON = 12

NSC = 2
NTILE = 16
NW = NSC * NTILE
ER = E // 128
CHS = 196
NP = 50176
RPS = NP // NTILE
NPR = NP // 16

_mesh = plsc.VectorSubcoreMesh(core_axis_name="c", subcore_axis_name="s")
_sc_params = pltpu.CompilerParams(use_tc_tiling_on_sc=False)


def _worker_span(wid):
    start = wid * ER // NW
    nrows = (wid + 1) * ER // NW - start
    return start, nrows


EB = 131072
EGRID = -(-E // EB)


def _split_body(idx_ref, row_ref, col_ref):
    row_ref[...] = idx_ref[0]
    col_ref[...] = idx_ref[1]


_split = pl.pallas_call(
    _split_body,
    grid=(EGRID,),
    in_specs=[pl.BlockSpec((2, EB), lambda i: (0, i))],
    out_specs=[
        pl.BlockSpec((EB,), lambda i: (i,)),
        pl.BlockSpec((EB,), lambda i: (i,)),
    ],
    out_shape=(
        jax.ShapeDtypeStruct((E,), I32),
        jax.ShapeDtypeStruct((E,), I32),
    ),
)

@functools.partial(
    pl.kernel,
    out_type=(
        jax.ShapeDtypeStruct((NP,), F32),
        jax.ShapeDtypeStruct((NP,), F32),
        jax.ShapeDtypeStruct((NP,), F32),
        jax.ShapeDtypeStruct((NP,), F32),
    ),
    mesh=_mesh,
    scratch_types=[
        pltpu.VMEM((CHS * 128,), I32),
        pltpu.VMEM((CHS * 128,), I32),
        pltpu.VMEM((CHS * 128,), F32),
        pltpu.VMEM((128,), F32),
        pltpu.VMEM((RPS,), F32),
        pltpu.VMEM_SHARED((NP,), F32),
        pltpu.VMEM_SHARED((NP,), F32),
        pltpu.SemaphoreType.DMA,
        pltpu.SemaphoreType.DMA,
    ],
    compiler_params=_sc_params,
)
def _degrees(row1, col1, w1, ones_h, z1_h, degw0, degw1, cnt0, cnt1,
             rowb, colb, wb, onesv, bounce, degw_s, cnt_s, sma, smb):
    cid = lax.axis_index("c")
    sid = lax.axis_index("s")
    wid = sid * NSC + cid
    off = sid * RPS
    pltpu.sync_copy(z1_h, bounce)
    pltpu.sync_copy(bounce, degw_s.at[pl.ds(off, RPS)])
    pltpu.sync_copy(bounce, cnt_s.at[pl.ds(off, RPS)])
    pltpu.sync_copy(ones_h, onesv)
    start, nrows = _worker_span(wid)
    pltpu.sync_copy(row1.at[pl.ds(start * 128, CHS * 128)], rowb)
    pltpu.sync_copy(col1.at[pl.ds(start * 128, CHS * 128)], colb)
    pltpu.sync_copy(w1.at[pl.ds(start * 128, CHS * 128)], wb)
    plsc.subcore_barrier()

    pltpu.async_copy(wb.at[pl.ds(0, 128)],
                     degw_s.at[rowb.at[pl.ds(0, 128)]], sma, add=True)
    pltpu.async_copy(onesv, cnt_s.at[colb.at[pl.ds(0, 128)]], smb, add=True)

    def body(j, carry):
        @pl.when(j + 1 < nrows)
        def _():
            pltpu.async_copy(wb.at[pl.ds((j + 1) * 128, 128)],
                             degw_s.at[rowb.at[pl.ds((j + 1) * 128, 128)]],
                             sma, add=True)
            pltpu.async_copy(onesv,
                             cnt_s.at[colb.at[pl.ds((j + 1) * 128, 128)]],
                             smb, add=True)

        pltpu.make_async_copy(wb.at[pl.ds(j * 128, 128)],
                              degw_s.at[rowb.at[pl.ds(j * 128, 128)]],
                              sma).wait()
        pltpu.make_async_copy(onesv,
                              cnt_s.at[colb.at[pl.ds(j * 128, 128)]],
                              smb).wait()
        return carry

    lax.fori_loop(0, nrows, body, 0)
    plsc.subcore_barrier()
    pltpu.sync_copy(degw_s.at[pl.ds(off, RPS)], bounce)

    @pl.when(cid == 0)
    def _():
        pltpu.sync_copy(bounce, degw0.at[pl.ds(off, RPS)])

    @pl.when(cid == 1)
    def _():
        pltpu.sync_copy(bounce, degw1.at[pl.ds(off, RPS)])

    pltpu.sync_copy(cnt_s.at[pl.ds(off, RPS)], bounce)

    @pl.when(cid == 0)
    def _():
        pltpu.sync_copy(bounce, cnt0.at[pl.ds(off, RPS)])

    @pl.when(cid == 1)
    def _():
        pltpu.sync_copy(bounce, cnt1.at[pl.ds(off, RPS)])


NB = 7168
GRID_N = NP // NB


def _xmean_body(x_ref, xm_ref):
    xm_ref[...] = jnp.sum(x_ref[...], axis=0) * (1.0 / T)


_xmean = pl.pallas_call(
    _xmean_body,
    grid=(GRID_N,),
    in_specs=[pl.BlockSpec((T, B, NB), lambda i: (0, 0, i))],
    out_specs=pl.BlockSpec((B, NB), lambda i: (0, i)),
    out_shape=jax.ShapeDtypeStruct((B, NP), F32),
)


def _scale_body(xm_ref, dwa, dwb, ca, cb, xs_ref, dinv_ref):
    iso = 1.0 / (dwa[...] + dwb[...] + 1e-3)
    dinv = lax.rsqrt(ca[...] + cb[...] + 1.0)
    dinv_ref[...] = dinv
    xs_ref[...] = xm_ref[...] * (iso * dinv)[None, :]


_scale = pl.pallas_call(
    _scale_body,
    grid=(GRID_N,),
    in_specs=[pl.BlockSpec((B, NB), lambda i: (0, i))]
    + [pl.BlockSpec((NB,), lambda i: (i,))] * 4,
    out_specs=[
        pl.BlockSpec((B, NB), lambda i: (0, i)),
        pl.BlockSpec((NB,), lambda i: (i,)),
    ],
    out_shape=(
        jax.ShapeDtypeStruct((B, NP), F32),
        jax.ShapeDtypeStruct((NP,), F32),
    ),
)


@functools.partial(
    pl.kernel,
    out_type=(
        jax.ShapeDtypeStruct((NP, B), F32),
        jax.ShapeDtypeStruct((NP, B), F32),
    ),
    mesh=_mesh,
    scratch_types=[
        pltpu.VMEM((CHS * 128,), I32),
        pltpu.VMEM((CHS * 128,), I32),
        pltpu.VMEM((RPS, B), F32),
    ] + [pltpu.VMEM((128, B), F32)] * 4
    + [pltpu.VMEM_SHARED((NP, B), F32),
        pltpu.VMEM_SHARED((NP, B), F32),
    ] + [pltpu.SemaphoreType.DMA] * 8,
    compiler_params=_sc_params,
)
def _edgepass(row1, col1, g_h, z8_h, acc0, acc1,
              rowb, colb, gbuf, g0, g1, g2, g3,
              g_s, acc_s, *sems):
    cid = lax.axis_index("c")
    sid = lax.axis_index("s")
    wid = sid * NSC + cid
    off = sid * RPS
    start, nrows = _worker_span(wid)
    pltpu.sync_copy(row1.at[pl.ds(start * 128, CHS * 128)], rowb)
    pltpu.sync_copy(col1.at[pl.ds(start * 128, CHS * 128)], colb)
    pltpu.sync_copy(g_h.at[pl.ds(off, RPS)], gbuf)
    pltpu.sync_copy(gbuf, g_s.at[pl.ds(off, RPS)])

    @pl.when(cid == 0)
    def _():
        pltpu.sync_copy(gbuf, acc_s.at[pl.ds(off, RPS)])

    @pl.when(cid == 1)
    def _():
        pltpu.sync_copy(z8_h, gbuf)
        pltpu.sync_copy(gbuf, acc_s.at[pl.ds(off, RPS)])

    plsc.subcore_barrier()

    gbufs = (g0, g1, g2, g3)
    sg = sems[:4]
    ss = sems[4:]
    NBUF = 4
    for i in range(NBUF):
        pltpu.async_copy(g_s.at[rowb.at[pl.ds(i * 128, 128)]], gbufs[i],
                         sg[i])

    def obody(k, carry):
        j = k * NBUF
        for i in range(NBUF):
            pltpu.make_async_copy(
                g_s.at[rowb.at[pl.ds((j + i) * 128, 128)]], gbufs[i],
                sg[i]).wait()
            pltpu.async_copy(
                gbufs[i], acc_s.at[colb.at[pl.ds((j + i) * 128, 128)]],
                ss[i], add=True)
        for i in range(NBUF):
            pltpu.make_async_copy(
                gbufs[i], acc_s.at[colb.at[pl.ds((j + i) * 128, 128)]],
                ss[i]).wait()
            jn = j + NBUF + i

            @pl.when(jn < nrows)
            def _(jn=jn, i=i):
                pltpu.async_copy(
                    g_s.at[rowb.at[pl.ds(jn * 128, 128)]], gbufs[i], sg[i])
        return carry

    lax.fori_loop(0, nrows // NBUF, obody, 0)

    tbase = nrows // NBUF * NBUF
    for i in range(NBUF - 1):
        jt = tbase + i

        @pl.when(jt < nrows)
        def _(jt=jt, i=i):
            pltpu.make_async_copy(
                g_s.at[rowb.at[pl.ds(jt * 128, 128)]], gbufs[i],
                sg[i]).wait()
            pltpu.sync_copy(
                gbufs[i], acc_s.at[colb.at[pl.ds(jt * 128, 128)]], add=True)
    plsc.subcore_barrier()
    pltpu.sync_copy(acc_s.at[pl.ds(off, RPS)], gbuf)

    @pl.when(cid == 0)
    def _():
        pltpu.sync_copy(gbuf, acc0.at[pl.ds(off, RPS)])

    @pl.when(cid == 1)
    def _():
        pltpu.sync_copy(gbuf, acc1.at[pl.ds(off, RPS)])


def _fin_body(ta, tb, dinv_ref, w_ref, b_ref, pw_ref, pb_ref, *out_refs):
    t8 = ta[...] + tb[...]
    eye = jnp.eye(B, dtype=F32)
    tt = lax.dot_general(eye, t8, (((1,), (1,)), ((), ())),
                         preferred_element_type=F32)
    tt = tt * dinv_ref[...][None, :]
    v = jnp.dot(w_ref[0, :], pw_ref[...], preferred_element_type=F32)
    c = jnp.dot(b_ref[...], pw_ref[...], preferred_element_type=F32) + pb_ref[...]
    for h in range(HORIZON):
        out_refs[h][...] = tt * v[h] + c[h]


_fin = pl.pallas_call(
    _fin_body,
    grid=(GRID_N,),
    in_specs=[
        pl.BlockSpec((NB, B), lambda i: (i, 0)),
        pl.BlockSpec((NB, B), lambda i: (i, 0)),
        pl.BlockSpec((NB,), lambda i: (i,)),
        pl.BlockSpec((1, 64), lambda i: (0, 0)),
        pl.BlockSpec((64,), lambda i: (0,)),
        pl.BlockSpec((64, HORIZON), lambda i: (0, 0)),
        pl.BlockSpec((HORIZON,), lambda i: (0,)),
    ],
    out_specs=[pl.BlockSpec((B, NB), lambda i: (0, i))] * HORIZON,
    out_shape=tuple(jax.ShapeDtypeStruct((B, NP), F32)
                    for _ in range(HORIZON)),
)


def kernel(x, edge_index, edge_weight, W, b, proj_W, proj_b):
    xt = jnp.transpose(x.reshape(B, N, T), (2, 0, 1))
    row1, col1 = _split(edge_index)
    ones_h = jnp.ones((128,), F32)
    z1 = jnp.zeros((RPS,), F32)
    z8 = jnp.zeros((RPS, B), F32)
    degw0, degw1, cnt0, cnt1 = _degrees(row1, col1, edge_weight, ones_h, z1)
    xm = _xmean(xt)
    xs, dinv = _scale(xm, degw0, degw1, cnt0, cnt1)
    g = jnp.transpose(xs)
    acc0, acc1 = _edgepass(row1, col1, g, z8)
    planes = _fin(acc0, acc1, dinv, W, b, proj_W, proj_b)
    return jnp.stack([p[:, :N] for p in planes], axis=-1)


# speedup vs baseline: 2.4222x; 1.0003x over previous
import functools

import jax
import jax.numpy as jnp
from jax import lax
from jax.experimental import pallas as pl
from jax.experimental.pallas import tpu as pltpu
from jax.experimental.pallas import tpu_sc as plsc

F32 = jnp.float32
I32 = jnp.int32

B, N, T, E = 8, 50000, 12, 800000
HORIZON = 12

NSC = 2
NTILE = 16
NW = NSC * NTILE
ER = E // 128
CHS = 196
NP = 50176
RPS = NP // NTILE

_mesh = plsc.VectorSubcoreMesh(core_axis_name="c", subcore_axis_name="s")
_sc_params = pltpu.CompilerParams(use_tc_tiling_on_sc=False)


def _worker_span(wid):
    start = wid * ER // NW
    nrows = (wid + 1) * ER // NW - start
    return start, nrows


EB = 131072
EGRID = -(-E // EB)


def _split_body(idx_ref, row_ref, col_ref):
    row_ref[...] = idx_ref[0]
    col_ref[...] = idx_ref[1]


_split = pl.pallas_call(
    _split_body,
    grid=(EGRID,),
    in_specs=[pl.BlockSpec((2, EB), lambda i: (0, i))],
    out_specs=[
        pl.BlockSpec((EB,), lambda i: (i,)),
        pl.BlockSpec((EB,), lambda i: (i,)),
    ],
    out_shape=(
        jax.ShapeDtypeStruct((E,), I32),
        jax.ShapeDtypeStruct((E,), I32),
    ),
)

@functools.partial(
    pl.kernel,
    out_type=(
        jax.ShapeDtypeStruct((NP,), F32),
        jax.ShapeDtypeStruct((NP,), F32),
        jax.ShapeDtypeStruct((NP,), F32),
        jax.ShapeDtypeStruct((NP,), F32),
    ),
    mesh=_mesh,
    scratch_types=[
        pltpu.VMEM((CHS * 128,), I32),
        pltpu.VMEM((CHS * 128,), I32),
        pltpu.VMEM((CHS * 128,), F32),
        pltpu.VMEM((128,), F32),
        pltpu.VMEM((RPS,), F32),
        pltpu.VMEM_SHARED((NP,), F32),
        pltpu.VMEM_SHARED((NP,), F32),
        pltpu.SemaphoreType.DMA,
        pltpu.SemaphoreType.DMA,
    ],
    compiler_params=_sc_params,
)
def _degrees(row1, col1, w1, ones_h, z1_h, degw0, degw1, cnt0, cnt1,
             rowb, colb, wb, onesv, bounce, degw_s, cnt_s, sma, smb):
    cid = lax.axis_index("c")
    sid = lax.axis_index("s")
    wid = sid * NSC + cid
    off = sid * RPS
    pltpu.sync_copy(z1_h, bounce)
    pltpu.sync_copy(bounce, degw_s.at[pl.ds(off, RPS)])
    pltpu.sync_copy(bounce, cnt_s.at[pl.ds(off, RPS)])
    pltpu.sync_copy(ones_h, onesv)
    start, nrows = _worker_span(wid)
    pltpu.sync_copy(row1.at[pl.ds(start * 128, CHS * 128)], rowb)
    pltpu.sync_copy(col1.at[pl.ds(start * 128, CHS * 128)], colb)
    pltpu.sync_copy(w1.at[pl.ds(start * 128, CHS * 128)], wb)
    plsc.subcore_barrier()

    pltpu.async_copy(wb.at[pl.ds(0, 128)],
                     degw_s.at[rowb.at[pl.ds(0, 128)]], sma, add=True)
    pltpu.async_copy(onesv, cnt_s.at[colb.at[pl.ds(0, 128)]], smb, add=True)

    def body(j, carry):
        @pl.when(j + 1 < nrows)
        def _():
            pltpu.async_copy(wb.at[pl.ds((j + 1) * 128, 128)],
                             degw_s.at[rowb.at[pl.ds((j + 1) * 128, 128)]],
                             sma, add=True)
            pltpu.async_copy(onesv,
                             cnt_s.at[colb.at[pl.ds((j + 1) * 128, 128)]],
                             smb, add=True)

        pltpu.make_async_copy(wb.at[pl.ds(j * 128, 128)],
                              degw_s.at[rowb.at[pl.ds(j * 128, 128)]],
                              sma).wait()
        pltpu.make_async_copy(onesv,
                              cnt_s.at[colb.at[pl.ds(j * 128, 128)]],
                              smb).wait()
        return carry

    lax.fori_loop(0, nrows, body, 0)
    plsc.subcore_barrier()
    pltpu.sync_copy(degw_s.at[pl.ds(off, RPS)], bounce)

    @pl.when(cid == 0)
    def _():
        pltpu.sync_copy(bounce, degw0.at[pl.ds(off, RPS)])

    @pl.when(cid == 1)
    def _():
        pltpu.sync_copy(bounce, degw1.at[pl.ds(off, RPS)])

    pltpu.sync_copy(cnt_s.at[pl.ds(off, RPS)], bounce)

    @pl.when(cid == 0)
    def _():
        pltpu.sync_copy(bounce, cnt0.at[pl.ds(off, RPS)])

    @pl.when(cid == 1)
    def _():
        pltpu.sync_copy(bounce, cnt1.at[pl.ds(off, RPS)])


NB = 7168
GRID_N = NP // NB


def _xmean_body(x_ref, xm_ref):
    xm_ref[...] = jnp.sum(x_ref[...], axis=0) * (1.0 / T)


_xmean = pl.pallas_call(
    _xmean_body,
    grid=(GRID_N,),
    in_specs=[pl.BlockSpec((T, B, NB), lambda i: (0, 0, i))],
    out_specs=pl.BlockSpec((B, NB), lambda i: (0, i)),
    out_shape=jax.ShapeDtypeStruct((B, NP), F32),
)


def _scale_body(xm_ref, dwa, dwb, ca, cb, xs_ref, dinv_ref):
    iso = 1.0 / (dwa[...] + dwb[...] + 1e-3)
    dinv = lax.rsqrt(ca[...] + cb[...] + 1.0)
    dinv_ref[...] = dinv
    xs_ref[...] = xm_ref[...] * (iso * dinv)[None, :]


_scale = pl.pallas_call(
    _scale_body,
    grid=(GRID_N,),
    in_specs=[pl.BlockSpec((B, NB), lambda i: (0, i))]
    + [pl.BlockSpec((NB,), lambda i: (i,))] * 4,
    out_specs=[
        pl.BlockSpec((B, NB), lambda i: (0, i)),
        pl.BlockSpec((NB,), lambda i: (i,)),
    ],
    out_shape=(
        jax.ShapeDtypeStruct((B, NP), F32),
        jax.ShapeDtypeStruct((NP,), F32),
    ),
)


@functools.partial(
    pl.kernel,
    out_type=(
        jax.ShapeDtypeStruct((NP, B), F32),
        jax.ShapeDtypeStruct((NP, B), F32),
    ),
    mesh=_mesh,
    scratch_types=[
        pltpu.VMEM((CHS * 128,), I32),
        pltpu.VMEM((CHS * 128,), I32),
        pltpu.VMEM((RPS, B), F32),
    ] + [pltpu.VMEM((128, B), F32)] * 4
    + [pltpu.VMEM_SHARED((NP, B), F32),
        pltpu.VMEM_SHARED((NP, B), F32),
    ] + [pltpu.SemaphoreType.DMA] * 8,
    compiler_params=_sc_params,
)
def _edgepass(row1, col1, g_h, z8_h, acc0, acc1,
              rowb, colb, gbuf, g0, g1, g2, g3,
              g_s, acc_s, *sems):
    cid = lax.axis_index("c")
    sid = lax.axis_index("s")
    wid = sid * NSC + cid
    off = sid * RPS
    start, nrows = _worker_span(wid)
    pltpu.sync_copy(row1.at[pl.ds(start * 128, CHS * 128)], rowb)
    pltpu.sync_copy(col1.at[pl.ds(start * 128, CHS * 128)], colb)
    pltpu.sync_copy(g_h.at[pl.ds(off, RPS)], gbuf)
    pltpu.sync_copy(gbuf, g_s.at[pl.ds(off, RPS)])

    @pl.when(cid == 0)
    def _():
        pltpu.sync_copy(gbuf, acc_s.at[pl.ds(off, RPS)])

    @pl.when(cid == 1)
    def _():
        pltpu.sync_copy(z8_h, gbuf)
        pltpu.sync_copy(gbuf, acc_s.at[pl.ds(off, RPS)])

    plsc.subcore_barrier()

    gbufs = (g0, g1, g2, g3)
    sg = sems[:4]
    ss = sems[4:]
    NBUF = 4
    for i in range(NBUF):
        pltpu.async_copy(g_s.at[rowb.at[pl.ds(i * 128, 128)]], gbufs[i],
                         sg[i])

    def obody(k, carry):
        j = k * NBUF
        for i in range(NBUF):
            pltpu.make_async_copy(
                g_s.at[rowb.at[pl.ds((j + i) * 128, 128)]], gbufs[i],
                sg[i]).wait()
            pltpu.async_copy(
                gbufs[i], acc_s.at[colb.at[pl.ds((j + i) * 128, 128)]],
                ss[i], add=True)
        for i in range(NBUF):
            pltpu.make_async_copy(
                gbufs[i], acc_s.at[colb.at[pl.ds((j + i) * 128, 128)]],
                ss[i]).wait()
            jn = j + NBUF + i

            @pl.when(jn < nrows)
            def _(jn=jn, i=i):
                pltpu.async_copy(
                    g_s.at[rowb.at[pl.ds(jn * 128, 128)]], gbufs[i], sg[i])
        return carry

    lax.fori_loop(0, nrows // NBUF, obody, 0)

    tbase = nrows // NBUF * NBUF
    for i in range(NBUF - 1):
        jt = tbase + i

        @pl.when(jt < nrows)
        def _(jt=jt, i=i):
            pltpu.make_async_copy(
                g_s.at[rowb.at[pl.ds(jt * 128, 128)]], gbufs[i],
                sg[i]).wait()
            pltpu.sync_copy(
                gbufs[i], acc_s.at[colb.at[pl.ds(jt * 128, 128)]], add=True)
    plsc.subcore_barrier()
    pltpu.sync_copy(acc_s.at[pl.ds(off, RPS)], gbuf)

    @pl.when(cid == 0)
    def _():
        pltpu.sync_copy(gbuf, acc0.at[pl.ds(off, RPS)])

    @pl.when(cid == 1)
    def _():
        pltpu.sync_copy(gbuf, acc1.at[pl.ds(off, RPS)])


def _fin_body(ta, tb, dinv_ref, w_ref, b_ref, pw_ref, pb_ref, *out_refs):
    t8 = ta[...] + tb[...]
    eye = jnp.eye(B, dtype=F32)
    tt = lax.dot_general(eye, t8, (((1,), (1,)), ((), ())),
                         preferred_element_type=F32)
    tt = tt * dinv_ref[...][None, :]
    v = jnp.dot(w_ref[0, :], pw_ref[...], preferred_element_type=F32)
    c = jnp.dot(b_ref[...], pw_ref[...], preferred_element_type=F32) + pb_ref[...]
    for h in range(HORIZON):
        out_refs[h][...] = tt * v[h] + c[h]


_fin = pl.pallas_call(
    _fin_body,
    grid=(GRID_N,),
    in_specs=[
        pl.BlockSpec((NB, B), lambda i: (i, 0)),
        pl.BlockSpec((NB, B), lambda i: (i, 0)),
        pl.BlockSpec((NB,), lambda i: (i,)),
        pl.BlockSpec((1, 64), lambda i: (0, 0)),
        pl.BlockSpec((64,), lambda i: (0,)),
        pl.BlockSpec((64, HORIZON), lambda i: (0, 0)),
        pl.BlockSpec((HORIZON,), lambda i: (0,)),
    ],
    out_specs=[pl.BlockSpec((B, NB), lambda i: (0, i))] * HORIZON,
    out_shape=tuple(jax.ShapeDtypeStruct((B, NP), F32)
                    for _ in range(HORIZON)),
)


def kernel(x, edge_index, edge_weight, W, b, proj_W, proj_b):
    xt = jnp.transpose(x.reshape(B, N, T), (2, 0, 1))
    row1, col1 = _split(edge_index)
    ones_h = jnp.ones((128,), F32)
    z1 = jnp.zeros((RPS,), F32)
    z8 = jnp.zeros((RPS, B), F32)
    degw0, degw1, cnt0, cnt1 = _degrees(row1, col1, edge_weight, ones_h, z1)
    xm = _xmean(xt)
    xs, dinv = _scale(xm, degw0, degw1, cnt0, cnt1)
    g = jnp.transpose(xs)
    acc0, acc1 = _edgepass(row1, col1, g, z8)
    planes = _fin(acc0, acc1, dinv, W, b, proj_W, proj_b)
    return jnp.stack([p[:, :N] for p in planes], axis=-1)
